# Initial kernel scaffold; baseline (speedup 1.0000x reference)
import jax
import jax.numpy as jnp
from jax.experimental import pallas as pl


def kernel(patch_features, queue):
    raise NotImplementedError("write your pallas kernel here")


# speedup vs baseline: 7.4307x; 7.4307x over previous
import functools

import jax
import jax.numpy as jnp
from jax.experimental import pallas as pl


def _body(a_ref, q_ref, o_ref, *, pblk):
    for p in range(pblk):
        a = a_ref[:, p, :]
        q = q_ref[p]
        sim = jax.lax.dot_general(
            a, q, (((1,), (1,)), ((), ())),
            preferred_element_type=jnp.float32)
        idx = jnp.argmax(sim, axis=1)
        m = sim.shape[1]
        onehot = (idx[:, None] == jax.lax.broadcasted_iota(jnp.int32, (1, m), 1)
                  ).astype(jnp.float32)
        o_ref[:, p, :] = jax.lax.dot_general(
            onehot, q, (((1,), (0,)), ((), ())),
            preferred_element_type=jnp.float32)


@jax.jit
def kernel(patch_features, queue):
    b, p_total, f = patch_features.shape
    _, m, _ = queue.shape
    pblk = 8
    grid = (p_total // pblk,)
    return pl.pallas_call(
        functools.partial(_body, pblk=pblk),
        grid=grid,
        in_specs=[
            pl.BlockSpec((b, pblk, f), lambda i: (0, i, 0)),
            pl.BlockSpec((pblk, m, f), lambda i: (i, 0, 0)),
        ],
        out_specs=pl.BlockSpec((b, pblk, f), lambda i: (0, i, 0)),
        out_shape=jax.ShapeDtypeStruct((b, p_total, f), jnp.float32),
    )(patch_features, queue)
